# Initial kernel scaffold; baseline (speedup 1.0000x reference)
import jax
import jax.numpy as jnp
from jax.experimental import pallas as pl


def kernel(bytes, emb, pos):
    raise NotImplementedError("write your pallas kernel here")


# speedup vs baseline: 1.3745x; 1.3745x over previous
import functools
import numpy as np
import jax
import jax.numpy as jnp
from jax import lax
from jax.experimental import pallas as pl
from jax.experimental.pallas import tpu as pltpu
from jax.experimental.pallas import tpu_sc as plsc

V = 256
D = 512
T = 2048
P = 16
K = 128
B = 4
N = B * T
L = 16


def _make_sc_kernel():
    info = plsc.get_sparse_core_info()
    NC, NS = info.num_cores, info.num_subcores
    NW = NC * NS
    rows_per_w = N // NW
    C = 64
    n_chunks = rows_per_w // C

    mesh = plsc.VectorSubcoreMesh(core_axis_name="c", subcore_axis_name="s")

    @functools.partial(
        pl.kernel,
        mesh=mesh,
        out_type=jax.ShapeDtypeStruct((N, D), jnp.float32),
        scratch_types=[
            pltpu.VMEM((C,), jnp.int32),
            pltpu.VMEM((C, D), jnp.float32),
            pltpu.VMEM((C, D), jnp.float32),
            pltpu.SemaphoreType.DMA,
        ],
    )
    def k(idx_hbm, emb_hbm, pos_hbm, out_hbm, idx_v, bufA, bufB, sem):
        cid = lax.axis_index("c")
        sid = lax.axis_index("s")
        wid = sid * NC + cid
        base = wid * rows_per_w
        tbase = base % T
        for c in range(n_chunks):
            r0 = base + c * C
            t0 = tbase + c * C
            pltpu.sync_copy(idx_hbm.at[pl.ds(r0, C)], idx_v)
            pltpu.sync_copy(pos_hbm.at[pl.ds(t0, C)], bufB)
            pltpu.async_copy(emb_hbm.at[idx_v], bufA, sem).wait()

            def add_row(r, _):
                for j in range(D // L):
                    sl = pl.ds(j * L, L)
                    plsc.addupdate(bufB.at[r, sl], bufA[r, sl])
                return _

            lax.fori_loop(0, C, add_row, 0)
            pltpu.sync_copy(bufB, out_hbm.at[pl.ds(r0, C)])

    return k


_sc_kernel = _make_sc_kernel()


def kernel(bytes, emb, pos):
    idx = bytes.reshape(N)
    out = _sc_kernel(idx, emb, pos)
    return out.reshape(B, K, P * D)


# speedup vs baseline: 1.5719x; 1.1436x over previous
import functools
import numpy as np
import jax
import jax.numpy as jnp
from jax import lax
from jax.experimental import pallas as pl
from jax.experimental.pallas import tpu as pltpu
from jax.experimental.pallas import tpu_sc as plsc

V = 256
D = 512
T = 2048
P = 16
K = 128
B = 4
N = B * T
L = 16


def _make_sc_kernel():
    info = plsc.get_sparse_core_info()
    NC, NS = info.num_cores, info.num_subcores
    NW = NC * NS
    rows_per_w = N // NW
    C = 32
    n_chunks = rows_per_w // C
    NBUF = 2

    mesh = plsc.VectorSubcoreMesh(core_axis_name="c", subcore_axis_name="s")

    @functools.partial(
        pl.kernel,
        mesh=mesh,
        out_type=jax.ShapeDtypeStruct((N, D), jnp.float32),
        scratch_types=[
            pltpu.VMEM((rows_per_w,), jnp.int32),
            pltpu.VMEM((NBUF, C, D), jnp.float32),
            pltpu.VMEM((NBUF, C, D), jnp.float32),
            pltpu.SemaphoreType.DMA,
            pltpu.SemaphoreType.DMA,
            pltpu.SemaphoreType.DMA,
            pltpu.SemaphoreType.DMA,
            pltpu.SemaphoreType.DMA,
            pltpu.SemaphoreType.DMA,
        ],
    )
    def k(idx_hbm, emb_hbm, pos_hbm, out_hbm,
          idx_v, bufA, bufB, sP0, sP1, sG0, sG1, sW0, sW1):
        semP = (sP0, sP1)
        semG = (sG0, sG1)
        semW = (sW0, sW1)
        cid = lax.axis_index("c")
        sid = lax.axis_index("s")
        wid = sid * NC + cid
        base = wid * rows_per_w
        tbase = base % T
        pltpu.sync_copy(idx_hbm.at[pl.ds(base, rows_per_w)], idx_v)

        hP = [None] * NBUF
        hG = [None] * NBUF
        hW = [None] * NBUF

        def start(c):
            p = c % NBUF
            if hW[p] is not None:
                hW[p].wait()
                hW[p] = None
            r0 = base + c * C
            t0 = tbase + c * C
            hP[p] = pltpu.async_copy(
                pos_hbm.at[pl.ds(t0, C)], bufB.at[p], semP[p])
            hG[p] = pltpu.async_copy(
                emb_hbm.at[idx_v.at[pl.ds(c * C, C)]], bufA.at[p], semG[p])

        start(0)
        for c in range(n_chunks):
            p = c % NBUF
            hP[p].wait()
            hG[p].wait()
            if c + 1 < n_chunks:
                start(c + 1)
            a = bufA.at[p]
            bb = bufB.at[p]

            def add_row(r, _):
                for j in range(D // L):
                    sl = pl.ds(j * L, L)
                    plsc.addupdate(bb.at[r, sl], a[r, sl])
                return _

            lax.fori_loop(0, C, add_row, 0)
            r0 = base + c * C
            hW[p] = pltpu.async_copy(
                bufB.at[p], out_hbm.at[pl.ds(r0, C)], semW[p])
        for p in range(NBUF):
            if hW[p] is not None:
                hW[p].wait()

    return k


_sc_kernel = _make_sc_kernel()


def kernel(bytes, emb, pos):
    idx = bytes.reshape(N)
    out = _sc_kernel(idx, emb, pos)
    return out.reshape(B, K, P * D)
